# Initial kernel scaffold; baseline (speedup 1.0000x reference)
import jax
import jax.numpy as jnp
from jax.experimental import pallas as pl


def kernel(node_embeddings, hyperedges, hyperedge_subsets, W, b):
    raise NotImplementedError("write your pallas kernel here")


# speedup vs baseline: 1.2440x; 1.2440x over previous
import jax
import jax.numpy as jnp
from jax import lax
from jax.experimental import pallas as pl
from jax.experimental.pallas import tpu as pltpu
from jax.experimental.pallas import tpu_sc as plsc

_N = 100000
_D = 128
_H = 10000
_GS = 32

_NC, _NS = 2, 16
_NW = _NC * _NS
_HPW = 320
_HPAD = _NW * _HPW
_CH = 4
_NCHUNK = _HPW // _CH
_ROWS = _CH * _GS
_NV = _D // 16


def _mm_body(ne_ref, wt_ref, b_ref, x_ref):
    x_ref[...] = jnp.maximum(
        jnp.dot(ne_ref[...], wt_ref[...], preferred_element_type=jnp.float32)
        + b_ref[...], 0.0)


def _transform(ne, wt, b):
    bn = 1000
    return pl.pallas_call(
        _mm_body,
        grid=(_N // bn,),
        in_specs=[
            pl.BlockSpec((bn, _D), lambda i: (i, 0)),
            pl.BlockSpec((_D, _D), lambda i: (0, 0)),
            pl.BlockSpec((1, _D), lambda i: (0, 0)),
        ],
        out_specs=pl.BlockSpec((bn, _D), lambda i: (i, 0)),
        out_shape=jax.ShapeDtypeStruct((_N, _D), jnp.float32),
    )(ne, wt, b.reshape(1, _D))


def _sc_body(x_hbm, idx_hbm, out_hbm, idx_v, buf, out_v, sem):
    wid = lax.axis_index("s") * _NC + lax.axis_index("c")
    pltpu.sync_copy(idx_hbm.at[wid], idx_v)

    def chunk(c, carry):
        pltpu.async_copy(x_hbm.at[idx_v.at[c]], buf, sem).wait()
        for h in range(_CH):
            accs = [jnp.zeros((16,), jnp.float32)] * _NV
            for r in range(_GS):
                for d in range(_NV):
                    accs[d] = accs[d] + buf[h * _GS + r, pl.ds(d * 16, 16)]
            row = c * _CH + h
            for d in range(_NV):
                out_v[row, pl.ds(d * 16, 16)] = accs[d] * (1.0 / _GS)
        return carry

    lax.fori_loop(0, _NCHUNK, chunk, 0)
    pltpu.sync_copy(out_v, out_hbm.at[pl.ds(wid * _HPW, _HPW)])


def _aggregate(x, idx):
    mesh = plsc.VectorSubcoreMesh(core_axis_name="c", subcore_axis_name="s")
    k = pl.kernel(
        _sc_body,
        out_type=jax.ShapeDtypeStruct((_HPAD, _D), jnp.float32),
        mesh=mesh,
        scratch_types=[
            pltpu.VMEM((_NCHUNK, _ROWS), jnp.int32),
            pltpu.VMEM((_ROWS, _D), jnp.float32),
            pltpu.VMEM((_HPW, _D), jnp.float32),
            pltpu.SemaphoreType.DMA,
        ],
    )
    return k(x, idx)


def kernel(node_embeddings, hyperedges, hyperedge_subsets, W, b):
    del hyperedges
    x = _transform(node_embeddings, W.T, b)
    idx = hyperedge_subsets.astype(jnp.int32).reshape(_H, _GS)
    idx = jnp.pad(idx, ((0, _HPAD - _H), (0, 0)))
    idx = idx.reshape(_NW, _NCHUNK, _ROWS)
    return _aggregate(x, idx)[:_H]
